# baseline (speedup 1.0000x reference)
import jax
import jax.numpy as jnp
from jax import lax
from jax.experimental import pallas as pl
from jax.experimental.pallas import tpu as pltpu

E = 8
TOPK = 2
LAYERS = 3
LOSS_COEF = 0.01
D = 768
T = 2048
BT = 512
NBLK = T // BT
EH = 4
NW = E // EH

_BF = jnp.bfloat16
_F32 = jnp.float32


def _dot(a, b):
    return jnp.dot(a.astype(_BF), b.astype(_BF),
                   preferred_element_type=_F32)


def _net_kernel(x_ref, enc_W_ref, enc_b_ref, gate_W_ref, Wc_ref, bc_ref,
                dec_W_ref, dec_b_ref, out_ref, loss_ref, imp_ref):
    i = pl.program_id(0)

    @pl.when(i == 0)
    def _init():
        imp_ref[...] = jnp.zeros_like(imp_ref)

    h = jnp.maximum(_dot(x_ref[:], enc_W_ref[:]) + enc_b_ref[:], 0.0)
    col = lax.broadcasted_iota(jnp.int32, (BT, E), 1)
    for l in range(LAYERS):
        hb = h.astype(_BF)
        logits = jnp.dot(hb, gate_W_ref[:].astype(_BF),
                         preferred_element_type=_F32)
        m0 = jnp.max(logits, axis=1, keepdims=True)
        i0 = jnp.min(jnp.where(logits == m0, col, E), axis=1, keepdims=True)
        masked = jnp.where(col == i0, -jnp.inf, logits)
        m1 = jnp.max(masked, axis=1, keepdims=True)
        i1 = jnp.min(jnp.where(masked == m1, col, E), axis=1, keepdims=True)
        u1 = jnp.exp(m1 - m0)
        den = 1.0 + u1
        g0 = 1.0 / den
        g1 = u1 / den
        gates = (jnp.where(col == i0, g0, 0.0)
                 + jnp.where(col == i1, g1, 0.0))
        gates_b = gates.astype(_BF).astype(_F32)
        acc = jnp.zeros((BT, D), _F32)
        for w in range(NW):
            z = jnp.dot(hb, Wc_ref[w].astype(_BF),
                        preferred_element_type=_F32) + bc_ref[w][None, :]
            zb = jnp.maximum(z, 0.0).astype(_BF).astype(_F32)
            for k in range(EH):
                e = w * EH + k
                acc = acc + gates_b[:, e:e + 1] * zb[:, k * D:(k + 1) * D]
        imp_ref[l:l + 1, :] += jnp.sum(gates, axis=0, keepdims=True)
        h = acc
    out_ref[:] = jnp.maximum(_dot(h, dec_W_ref[:]) + dec_b_ref[:], 0.0)

    @pl.when(i == NBLK - 1)
    def _fin():
        loss = jnp.float32(0.0)
        for l in range(LAYERS):
            imp = imp_ref[l:l + 1, :]
            mean = jnp.mean(imp)
            var = jnp.mean((imp - mean) ** 2)
            loss = loss + LOSS_COEF * var / (mean * mean + 1e-10)
        loss_ref[:, :] = jnp.broadcast_to(loss, (1, 1))


@jax.jit
def kernel(x, enc_W, enc_b, gate_W, We, be, dec_W, dec_b):
    Wc = We.transpose(1, 0, 2).reshape(D, E * D)
    Wc = Wc.reshape(D, NW, EH * D).transpose(1, 0, 2)
    bc = be.reshape(NW, EH * D)
    full = lambda *s: pl.BlockSpec(s, lambda i: (0,) * len(s))
    out, loss = pl.pallas_call(
        _net_kernel,
        grid=(NBLK,),
        in_specs=[
            pl.BlockSpec((BT, D), lambda i: (i, 0)),
            full(D, D),
            full(1, D),
            full(D, E),
            full(NW, D, EH * D),
            full(NW, EH * D),
            full(D, D),
            full(1, D),
        ],
        out_specs=(
            pl.BlockSpec((BT, D), lambda i: (i, 0)),
            pl.BlockSpec((1, 1), lambda i: (0, 0)),
        ),
        out_shape=(
            jax.ShapeDtypeStruct((T, D), jnp.float32),
            jax.ShapeDtypeStruct((1, 1), jnp.float32),
        ),
        scratch_shapes=[pltpu.VMEM((LAYERS, E), jnp.float32)],
        compiler_params=pltpu.CompilerParams(
            dimension_semantics=("arbitrary",),
            vmem_limit_bytes=100 * 1024 * 1024,
        ),
    )(x, enc_W, enc_b.reshape(1, D), gate_W, Wc, bc,
      dec_W, dec_b.reshape(1, D))
    return out, loss.reshape(())


# speedup vs baseline: 1.5753x; 1.5753x over previous
import jax
import jax.numpy as jnp
from jax import lax
from jax.experimental import pallas as pl
from jax.experimental.pallas import tpu as pltpu

E = 8
TOPK = 2
LAYERS = 3
LOSS_COEF = 0.01
D = 768
T = 2048
BT = 1024
NBLK = T // BT
EH = 4
NW = E // EH

_BF = jnp.bfloat16
_F32 = jnp.float32


def _dot(a, b):
    return jnp.dot(a.astype(_BF), b.astype(_BF),
                   preferred_element_type=_F32)


def _net_kernel(x_ref, enc_W_ref, enc_b_ref, gate_W_ref, Wc_ref, bc_ref,
                dec_W_ref, dec_b_ref, out_ref, loss_ref, imp_ref,
                We_vmem, We_sem):
    i = pl.program_id(0)

    @pl.when(i == 0)
    def _init():
        imp_ref[...] = jnp.zeros_like(imp_ref)
        for e in range(E):
            pltpu.make_async_copy(Wc_ref.at[e], We_vmem.at[e],
                                  We_sem.at[e]).start()

    h = jnp.maximum(_dot(x_ref[:], enc_W_ref[:]) + enc_b_ref[:], 0.0)
    col = lax.broadcasted_iota(jnp.int32, (BT, E), 1)
    for l in range(LAYERS):
        hb = h.astype(_BF)
        logits = jnp.dot(hb, gate_W_ref[:].astype(_BF),
                         preferred_element_type=_F32)
        m0 = jnp.max(logits, axis=1, keepdims=True)
        i0 = jnp.min(jnp.where(logits == m0, col, E), axis=1, keepdims=True)
        masked = jnp.where(col == i0, -jnp.inf, logits)
        m1 = jnp.max(masked, axis=1, keepdims=True)
        i1 = jnp.min(jnp.where(masked == m1, col, E), axis=1, keepdims=True)
        u1 = jnp.exp(m1 - m0)
        den = 1.0 + u1
        g0 = 1.0 / den
        g1 = u1 / den
        gates = (jnp.where(col == i0, g0, 0.0)
                 + jnp.where(col == i1, g1, 0.0))
        gates_b = gates.astype(_BF).astype(_F32)
        acc = jnp.zeros((BT, D), _F32)
        for e in range(E):
            if l == 0:
                @pl.when(i == 0)
                def _wait(e=e):
                    pltpu.make_async_copy(Wc_ref.at[e], We_vmem.at[e],
                                          We_sem.at[e]).wait()
            he = jnp.maximum(jnp.dot(hb, We_vmem[e].astype(_BF),
                                     preferred_element_type=_F32)
                             + bc_ref[e][None, :], 0.0)
            acc = acc + gates_b[:, e:e + 1] * he.astype(_BF).astype(_F32)
        imp_ref[l:l + 1, :] += jnp.sum(gates, axis=0, keepdims=True)
        h = acc
    out_ref[:] = jnp.maximum(_dot(h, dec_W_ref[:]) + dec_b_ref[:], 0.0)

    @pl.when(i == NBLK - 1)
    def _fin():
        loss = jnp.float32(0.0)
        for l in range(LAYERS):
            imp = imp_ref[l:l + 1, :]
            mean = jnp.mean(imp)
            var = jnp.mean((imp - mean) ** 2)
            loss = loss + LOSS_COEF * var / (mean * mean + 1e-10)
        loss_ref[:, :] = jnp.broadcast_to(loss, (1, 1))


@jax.jit
def kernel(x, enc_W, enc_b, gate_W, We, be, dec_W, dec_b):
    Wc = We
    bc = be
    full = lambda *s: pl.BlockSpec(s, lambda i: (0,) * len(s))
    out, loss = pl.pallas_call(
        _net_kernel,
        grid=(NBLK,),
        in_specs=[
            pl.BlockSpec((BT, D), lambda i: (i, 0)),
            full(D, D),
            full(1, D),
            full(D, E),
            pl.BlockSpec(memory_space=pltpu.MemorySpace.HBM),
            full(E, D),
            full(D, D),
            full(1, D),
        ],
        out_specs=(
            pl.BlockSpec((BT, D), lambda i: (i, 0)),
            pl.BlockSpec((1, 1), lambda i: (0, 0)),
        ),
        out_shape=(
            jax.ShapeDtypeStruct((T, D), jnp.float32),
            jax.ShapeDtypeStruct((1, 1), jnp.float32),
        ),
        scratch_shapes=[pltpu.VMEM((LAYERS, E), jnp.float32),
                        pltpu.VMEM((E, D, D), jnp.float32),
                        pltpu.SemaphoreType.DMA((E,))],
        compiler_params=pltpu.CompilerParams(
            dimension_semantics=("arbitrary",),
            vmem_limit_bytes=100 * 1024 * 1024,
        ),
    )(x, enc_W, enc_b.reshape(1, D), gate_W, Wc, bc,
      dec_W, dec_b.reshape(1, D))
    return out, loss.reshape(())


# speedup vs baseline: 1.6238x; 1.0308x over previous
import jax
import jax.numpy as jnp
from jax import lax
from jax.experimental import pallas as pl
from jax.experimental.pallas import tpu as pltpu

E = 8
TOPK = 2
LAYERS = 3
LOSS_COEF = 0.01
D = 768
T = 2048
BT = 1024
NBLK = T // BT
EH = 4
NW = E // EH

_BF = jnp.bfloat16
_F32 = jnp.float32


def _dot(a, b):
    return jnp.dot(a.astype(_BF), b.astype(_BF),
                   preferred_element_type=_F32)


def _net_kernel(x_ref, enc_W_ref, enc_b_ref, gate_W_ref, Wc_ref, bc_ref,
                dec_W_ref, dec_b_ref, out_ref, loss_ref, imp_ref):
    i = pl.program_id(0)

    @pl.when(i == 0)
    def _init():
        imp_ref[...] = jnp.zeros_like(imp_ref)

    h = jnp.maximum(_dot(x_ref[:], enc_W_ref[:]) + enc_b_ref[:], 0.0)
    col = lax.broadcasted_iota(jnp.int32, (BT, E), 1)
    for l in range(LAYERS):
        hb = h.astype(_BF)
        logits = jnp.dot(hb, gate_W_ref[:].astype(_BF),
                         preferred_element_type=_F32)
        m0 = jnp.max(logits, axis=1, keepdims=True)
        i0 = jnp.min(jnp.where(logits == m0, col, E), axis=1, keepdims=True)
        masked = jnp.where(col == i0, -jnp.inf, logits)
        m1 = jnp.max(masked, axis=1, keepdims=True)
        i1 = jnp.min(jnp.where(masked == m1, col, E), axis=1, keepdims=True)
        u1 = jnp.exp(m1 - m0)
        den = 1.0 + u1
        g0 = 1.0 / den
        g1 = u1 / den
        gates = (jnp.where(col == i0, g0, 0.0)
                 + jnp.where(col == i1, g1, 0.0))
        gates_b = gates.astype(_BF).astype(_F32)
        acc = jnp.zeros((BT, D), _F32)
        for e in range(E):
            he = jnp.maximum(jnp.dot(hb, Wc_ref[e].astype(_BF),
                                     preferred_element_type=_F32)
                             + bc_ref[e][None, :], 0.0)
            acc = acc + gates_b[:, e:e + 1] * he.astype(_BF).astype(_F32)
        imp_ref[l:l + 1, :] += jnp.sum(gates, axis=0, keepdims=True)
        h = acc
    out_ref[:] = jnp.maximum(_dot(h, dec_W_ref[:]) + dec_b_ref[:], 0.0)

    @pl.when(i == NBLK - 1)
    def _fin():
        loss = jnp.float32(0.0)
        for l in range(LAYERS):
            imp = imp_ref[l:l + 1, :]
            mean = jnp.mean(imp)
            var = jnp.mean((imp - mean) ** 2)
            loss = loss + LOSS_COEF * var / (mean * mean + 1e-10)
        loss_ref[:, :] = jnp.broadcast_to(loss, (1, 1))


@jax.jit
def kernel(x, enc_W, enc_b, gate_W, We, be, dec_W, dec_b):
    Wc = We
    bc = be
    full = lambda *s: pl.BlockSpec(s, lambda i: (0,) * len(s))
    out, loss = pl.pallas_call(
        _net_kernel,
        grid=(NBLK,),
        in_specs=[
            pl.BlockSpec((BT, D), lambda i: (i, 0)),
            full(D, D),
            full(1, D),
            full(D, E),
            full(E, D, D),
            full(E, D),
            full(D, D),
            full(1, D),
        ],
        out_specs=(
            pl.BlockSpec((BT, D), lambda i: (i, 0)),
            pl.BlockSpec((1, 1), lambda i: (0, 0)),
        ),
        out_shape=(
            jax.ShapeDtypeStruct((T, D), jnp.float32),
            jax.ShapeDtypeStruct((1, 1), jnp.float32),
        ),
        scratch_shapes=[pltpu.VMEM((LAYERS, E), jnp.float32)],
        compiler_params=pltpu.CompilerParams(
            dimension_semantics=("arbitrary",),
            vmem_limit_bytes=100 * 1024 * 1024,
        ),
    )(x, enc_W, enc_b.reshape(1, D), gate_W, Wc, bc,
      dec_W, dec_b.reshape(1, D))
    return out, loss.reshape(())
